# baseline (speedup 1.0000x reference)
import functools

import jax
import jax.numpy as jnp
from jax import lax
from jax.experimental import pallas as pl
from jax.experimental.pallas import tpu as pltpu
from jax.experimental.pallas import tpu_sc as plsc

N = 10000
E = 160000
CHUNK = 128
NC, NS = 2, 16
E_PAD = 163840
DUMP = 10008
ROWS_PER_TILE = 632
SP_ROWS = NS * ROWS_PER_TILE
DEG_CHUNKS = E_PAD // (NC * NS * CHUNK)
AGG_CHUNKS = E_PAD // (NS * CHUNK)

_mesh = plsc.VectorSubcoreMesh(core_axis_name="c", subcore_axis_name="s")


DEG_W = 128
DEG_NBUF = 4
NBUF = 2
IDX_AHEAD = 4
_DO_SCATTER = False


@functools.partial(
    pl.kernel,
    out_type=jax.ShapeDtypeStruct((NC * SP_ROWS, DEG_W), jnp.float32),
    mesh=_mesh,
    scratch_types=[
        pltpu.VMEM_SHARED((SP_ROWS, DEG_W), jnp.float32),
        pltpu.VMEM((DEG_CHUNKS, CHUNK), jnp.int32),
        pltpu.VMEM((CHUNK, DEG_W), jnp.float32),
        pltpu.SemaphoreType.DMA((DEG_NBUF,)),
    ],
)
def _deg_kernel(dst_hbm, ones_hbm, zeros_hbm, out_hbm, acc_sh, didx_all, ones_v, ssem):
    c = lax.axis_index("c")
    s = lax.axis_index("s")
    wid = c * NS + s
    pltpu.sync_copy(zeros_hbm, acc_sh.at[pl.ds(s * ROWS_PER_TILE, ROWS_PER_TILE)])
    pltpu.sync_copy(ones_hbm, ones_v)
    pltpu.sync_copy(dst_hbm.at[pl.ds(wid * DEG_CHUNKS, DEG_CHUNKS)], didx_all)
    plsc.subcore_barrier()

    sd = {}
    for k in range(DEG_CHUNKS):
        if k - DEG_NBUF >= 0:
            sd[k - DEG_NBUF].wait()
        sd[k] = pltpu.async_copy(
            ones_v, acc_sh.at[didx_all.at[k]], ssem.at[k % DEG_NBUF], add=True)
    for k in range(DEG_CHUNKS - DEG_NBUF, DEG_CHUNKS):
        sd[k].wait()
    plsc.subcore_barrier()
    pltpu.sync_copy(
        acc_sh.at[pl.ds(s * ROWS_PER_TILE, ROWS_PER_TILE)],
        out_hbm.at[pl.ds(c * SP_ROWS + s * ROWS_PER_TILE, ROWS_PER_TILE)],
    )


@functools.partial(
    pl.kernel,
    out_type=jax.ShapeDtypeStruct((NC * SP_ROWS, 128), jnp.float32),
    mesh=_mesh,
    scratch_types=[
        pltpu.VMEM_SHARED((SP_ROWS, 128), jnp.float32),
        pltpu.VMEM((AGG_CHUNKS, CHUNK), jnp.int32),
        [pltpu.VMEM((CHUNK,), jnp.int32) for _ in range(IDX_AHEAD)],
        pltpu.VMEM((NBUF * CHUNK, 128), jnp.float32),
        pltpu.SemaphoreType.DMA((IDX_AHEAD,)),
        pltpu.SemaphoreType.DMA((NBUF,)),
        pltpu.SemaphoreType.DMA((NBUF,)),
    ],
)
def _agg_kernel(xs_hbm, src_hbm, dst_hbm, zeros_hbm, out_hbm,
                acc_sh, didx_all, sidx, rows_v, isem, gsem, ssem):
    c = lax.axis_index("c")
    s = lax.axis_index("s")
    pltpu.sync_copy(zeros_hbm, acc_sh.at[pl.ds(s * ROWS_PER_TILE, ROWS_PER_TILE)])
    pltpu.sync_copy(dst_hbm.at[pl.ds(s * AGG_CHUNKS, AGG_CHUNKS)], didx_all)
    plsc.subcore_barrier()
    src_base = (c * NS + s) * AGG_CHUNKS

    def _gather(k):
        return pltpu.async_copy(
            xs_hbm.at[sidx[k % IDX_AHEAD]],
            rows_v.at[pl.ds((k % NBUF) * CHUNK, CHUNK)], gsem.at[k % NBUF])

    def _scatter(k):
        return pltpu.async_copy(
            rows_v.at[pl.ds((k % NBUF) * CHUNK, CHUNK)],
            acc_sh.at[didx_all.at[k]], ssem.at[k % NBUF], add=True)

    idn = {j: pltpu.async_copy(src_hbm.at[src_base + j], sidx[j % IDX_AHEAD],
                               isem.at[j % IDX_AHEAD])
           for j in range(IDX_AHEAD)}
    idn[0].wait()
    gd = {0: _gather(0)}
    sd = {}
    for k in range(AGG_CHUNKS):
        if k + 1 < AGG_CHUNKS:
            if k - 1 >= 0 and _DO_SCATTER:
                sd[k - 1].wait()
            idn[k + 1].wait()
            gd[k + 1] = _gather(k + 1)
        gd[k].wait()
        if _DO_SCATTER:
            sd[k] = _scatter(k)
        j = k + IDX_AHEAD
        if j < AGG_CHUNKS:
            idn[j] = pltpu.async_copy(src_hbm.at[src_base + j],
                                      sidx[j % IDX_AHEAD],
                                      isem.at[j % IDX_AHEAD])
    if _DO_SCATTER:
        sd[AGG_CHUNKS - 2].wait()
        sd[AGG_CHUNKS - 1].wait()
    plsc.subcore_barrier()
    pltpu.sync_copy(
        acc_sh.at[pl.ds(s * ROWS_PER_TILE, ROWS_PER_TILE)],
        out_hbm.at[pl.ds(c * SP_ROWS + s * ROWS_PER_TILE, ROWS_PER_TILE)],
    )


BLK = 1000


def _prep_body(d0_ref, d1_ref, x_ref, dis_ref, xs_ref):
    deg = d0_ref[:, :1] + d1_ref[:, :1] + 1.0
    dis = lax.rsqrt(deg)
    dis_ref[...] = jnp.broadcast_to(dis, dis_ref.shape)
    xs_ref[...] = x_ref[...] * dis


def _mm_body(a1_ref, xs_ref, dis_ref, w1_ref, b1_ref, w2_ref, ts_ref):
    p = (a1_ref[...] + xs_ref[...]) * dis_ref[...]
    h = jnp.dot(p, w1_ref[...], preferred_element_type=jnp.float32) + b1_ref[...]
    h = jnp.maximum(h, 0.0)
    ts_ref[...] = (
        jnp.dot(h, w2_ref[...], preferred_element_type=jnp.float32) * dis_ref[...]
    )


def _post_body(a2_ref, ts_ref, dis_ref, b2_ref, out_ref):
    out_ref[...] = (a2_ref[...] + ts_ref[...]) * dis_ref[...] + b2_ref[...]


def _row_spec(cols):
    return pl.BlockSpec((BLK, cols), lambda i: (i, 0))


def _full_spec(r, c):
    return pl.BlockSpec((r, c), lambda i: (0, 0))


def kernel(x, edge_index, W1, b1, W2, b2):
    src = edge_index[0].astype(jnp.int32)
    dst = edge_index[1].astype(jnp.int32)
    pad = E_PAD - E
    src_p = jnp.concatenate([src, jnp.zeros((pad,), jnp.int32)])
    dst_p = jnp.concatenate([dst, jnp.full((pad,), DUMP, jnp.int32)])
    dst_rows = dst_p.reshape(E_PAD // CHUNK, CHUNK)
    src_rows = jnp.concatenate([src_p, src_p + N]).reshape(2 * E_PAD // CHUNK, CHUNK)
    onesw = jnp.ones((CHUNK, DEG_W), jnp.float32)
    zw = jnp.zeros((ROWS_PER_TILE, DEG_W), jnp.float32)
    z128 = jnp.zeros((ROWS_PER_TILE, 128), jnp.float32)

    degp = _deg_kernel(dst_rows, onesw, zw)
    d0 = degp[:N, :16]
    d1 = degp[SP_ROWS:SP_ROWS + N, :16]

    grid = (N // BLK,)
    dis, xs = pl.pallas_call(
        _prep_body,
        grid=grid,
        in_specs=[_row_spec(16), _row_spec(16), _row_spec(256)],
        out_specs=[_row_spec(256), _row_spec(256)],
        out_shape=[
            jax.ShapeDtypeStruct((N, 256), jnp.float32),
            jax.ShapeDtypeStruct((N, 256), jnp.float32),
        ],
    )(d0, d1, x)

    xs2 = jnp.concatenate([xs[:, :128], xs[:, 128:]], axis=0)
    a1p = _agg_kernel(xs2, src_rows, dst_rows, z128)
    a1 = jnp.concatenate([a1p[:N], a1p[SP_ROWS:SP_ROWS + N]], axis=1)

    ts = pl.pallas_call(
        _mm_body,
        grid=grid,
        in_specs=[
            _row_spec(256), _row_spec(256), _row_spec(256),
            _full_spec(256, 512), _full_spec(1, 512), _full_spec(512, 256),
        ],
        out_specs=_row_spec(256),
        out_shape=jax.ShapeDtypeStruct((N, 256), jnp.float32),
    )(a1, xs, dis, W1, b1.reshape(1, 512), W2)

    ts2 = jnp.concatenate([ts[:, :128], ts[:, 128:]], axis=0)
    a2p = _agg_kernel(ts2, src_rows, dst_rows, z128)
    a2 = jnp.concatenate([a2p[:N], a2p[SP_ROWS:SP_ROWS + N]], axis=1)

    out = pl.pallas_call(
        _post_body,
        grid=grid,
        in_specs=[
            _row_spec(256), _row_spec(256), _row_spec(256), _full_spec(1, 256),
        ],
        out_specs=_row_spec(256),
        out_shape=jax.ShapeDtypeStruct((N, 256), jnp.float32),
    )(a2, ts, dis, b2.reshape(1, 256))
    return out


# speedup vs baseline: 1.9270x; 1.9270x over previous
import functools

import jax
import jax.numpy as jnp
from jax import lax
from jax.experimental import pallas as pl
from jax.experimental.pallas import tpu as pltpu
from jax.experimental.pallas import tpu_sc as plsc

N = 10000
E = 160000
CHUNK = 128
NC, NS = 2, 16
E_PAD = 163840
DUMP = 10000
ROWS_PER_TILE = 632
SP_ROWS = NS * ROWS_PER_TILE
DEG_CHUNKS = E_PAD // (NC * NS * CHUNK)
AGG_CHUNKS = E_PAD // (NS * CHUNK)

_mesh = plsc.VectorSubcoreMesh(core_axis_name="c", subcore_axis_name="s")


DEG_W = 128
DEG_NBUF = 4
NBUF = 2
IDX_AHEAD = 4
_DO_SCATTER = True


@functools.partial(
    pl.kernel,
    out_type=jax.ShapeDtypeStruct((NC * SP_ROWS, DEG_W), jnp.float32),
    mesh=_mesh,
    scratch_types=[
        pltpu.VMEM_SHARED((SP_ROWS, DEG_W), jnp.float32),
        pltpu.VMEM((DEG_CHUNKS, CHUNK), jnp.int32),
        pltpu.VMEM((CHUNK, DEG_W), jnp.float32),
        pltpu.SemaphoreType.DMA((DEG_NBUF,)),
    ],
)
def _deg_kernel(dst_hbm, ones_hbm, zeros_hbm, out_hbm, acc_sh, didx_all, ones_v, ssem):
    c = lax.axis_index("c")
    s = lax.axis_index("s")
    wid = c * NS + s
    pltpu.sync_copy(zeros_hbm, acc_sh.at[pl.ds(s * ROWS_PER_TILE, ROWS_PER_TILE)])
    pltpu.sync_copy(ones_hbm, ones_v)
    pltpu.sync_copy(dst_hbm.at[pl.ds(wid * DEG_CHUNKS, DEG_CHUNKS)], didx_all)
    plsc.subcore_barrier()

    sd = {}
    for k in range(DEG_CHUNKS):
        if k - DEG_NBUF >= 0:
            sd[k - DEG_NBUF].wait()
        sd[k] = pltpu.async_copy(
            ones_v, acc_sh.at[didx_all.at[k]], ssem.at[k % DEG_NBUF], add=True)
    for k in range(DEG_CHUNKS - DEG_NBUF, DEG_CHUNKS):
        sd[k].wait()
    plsc.subcore_barrier()
    pltpu.sync_copy(
        acc_sh.at[pl.ds(s * ROWS_PER_TILE, ROWS_PER_TILE)],
        out_hbm.at[pl.ds(c * SP_ROWS + s * ROWS_PER_TILE, ROWS_PER_TILE)],
    )


@functools.partial(
    pl.kernel,
    out_type=jax.ShapeDtypeStruct((NC * SP_ROWS, 128), jnp.float32),
    mesh=_mesh,
    scratch_types=[
        pltpu.VMEM_SHARED((SP_ROWS, 128), jnp.float32),
        pltpu.VMEM((AGG_CHUNKS, CHUNK), jnp.int32),
        [pltpu.VMEM((CHUNK,), jnp.int32) for _ in range(IDX_AHEAD)],
        pltpu.VMEM((NBUF * CHUNK, 128), jnp.float32),
        pltpu.SemaphoreType.DMA((IDX_AHEAD,)),
        pltpu.SemaphoreType.DMA((NBUF,)),
        pltpu.SemaphoreType.DMA((NBUF,)),
    ],
)
def _agg_kernel(xs_hbm, src_hbm, dst_hbm, zeros_hbm, out_hbm,
                acc_sh, didx_all, sidx, rows_v, isem, gsem, ssem):
    c = lax.axis_index("c")
    s = lax.axis_index("s")
    pltpu.sync_copy(zeros_hbm, acc_sh.at[pl.ds(s * ROWS_PER_TILE, ROWS_PER_TILE)])
    pltpu.sync_copy(dst_hbm.at[pl.ds(s * AGG_CHUNKS, AGG_CHUNKS)], didx_all)
    plsc.subcore_barrier()
    src_base = (c * NS + s) * AGG_CHUNKS

    def _gather(k):
        return pltpu.async_copy(
            xs_hbm.at[sidx[k % IDX_AHEAD]],
            rows_v.at[pl.ds((k % NBUF) * CHUNK, CHUNK)], gsem.at[k % NBUF])

    def _scatter(k):
        return pltpu.async_copy(
            rows_v.at[pl.ds((k % NBUF) * CHUNK, CHUNK)],
            acc_sh.at[didx_all.at[k]], ssem.at[k % NBUF], add=True)

    idn = {j: pltpu.async_copy(src_hbm.at[src_base + j], sidx[j % IDX_AHEAD],
                               isem.at[j % IDX_AHEAD])
           for j in range(IDX_AHEAD)}
    idn[0].wait()
    gd = {0: _gather(0)}
    sd = {}
    for k in range(AGG_CHUNKS):
        if k + 1 < AGG_CHUNKS:
            if k - 1 >= 0 and _DO_SCATTER:
                sd[k - 1].wait()
            idn[k + 1].wait()
            gd[k + 1] = _gather(k + 1)
        gd[k].wait()
        if _DO_SCATTER:
            sd[k] = _scatter(k)
        j = k + IDX_AHEAD
        if j < AGG_CHUNKS:
            idn[j] = pltpu.async_copy(src_hbm.at[src_base + j],
                                      sidx[j % IDX_AHEAD],
                                      isem.at[j % IDX_AHEAD])
    if _DO_SCATTER:
        sd[AGG_CHUNKS - 2].wait()
        sd[AGG_CHUNKS - 1].wait()
    plsc.subcore_barrier()
    pltpu.sync_copy(
        acc_sh.at[pl.ds(s * ROWS_PER_TILE, ROWS_PER_TILE)],
        out_hbm.at[pl.ds(c * SP_ROWS + s * ROWS_PER_TILE, ROWS_PER_TILE)],
    )


BLK = 1000


def _prep_body(d0_ref, d1_ref, x_ref, dis_ref, xs_ref):
    deg = d0_ref[:, :1] + d1_ref[:, :1] + 1.0
    dis = lax.rsqrt(deg)
    dis_ref[...] = jnp.broadcast_to(dis, dis_ref.shape)
    xs_ref[...] = x_ref[...] * dis


def _mm_body(a1_ref, xs_ref, dis_ref, w1_ref, b1_ref, w2_ref, ts_ref):
    p = (a1_ref[...] + xs_ref[...]) * dis_ref[...]
    h = jnp.dot(p, w1_ref[...], preferred_element_type=jnp.float32) + b1_ref[...]
    h = jnp.maximum(h, 0.0)
    ts_ref[...] = (
        jnp.dot(h, w2_ref[...], preferred_element_type=jnp.float32) * dis_ref[...]
    )


def _post_body(a2_ref, ts_ref, dis_ref, b2_ref, out_ref):
    out_ref[...] = (a2_ref[...] + ts_ref[...]) * dis_ref[...] + b2_ref[...]


def _row_spec(cols):
    return pl.BlockSpec((BLK, cols), lambda i: (i, 0))


def _full_spec(r, c):
    return pl.BlockSpec((r, c), lambda i: (0, 0))


def kernel(x, edge_index, W1, b1, W2, b2):
    src = edge_index[0].astype(jnp.int32)
    dst = edge_index[1].astype(jnp.int32)
    pad = E_PAD - E
    pad_idx = jnp.arange(pad, dtype=jnp.int32)
    src_p = jnp.concatenate([src, pad_idx % N])
    dst_p = jnp.concatenate([dst, DUMP + pad_idx % (SP_ROWS - DUMP)])
    dst_rows = dst_p.reshape(E_PAD // CHUNK, CHUNK)
    src_rows = jnp.concatenate([src_p, src_p + N]).reshape(2 * E_PAD // CHUNK, CHUNK)
    onesw = jnp.ones((CHUNK, DEG_W), jnp.float32)
    zw = jnp.zeros((ROWS_PER_TILE, DEG_W), jnp.float32)
    z128 = jnp.zeros((ROWS_PER_TILE, 128), jnp.float32)

    degp = _deg_kernel(dst_rows, onesw, zw)
    d0 = degp[:N, :16]
    d1 = degp[SP_ROWS:SP_ROWS + N, :16]

    grid = (N // BLK,)
    dis, xs = pl.pallas_call(
        _prep_body,
        grid=grid,
        in_specs=[_row_spec(16), _row_spec(16), _row_spec(256)],
        out_specs=[_row_spec(256), _row_spec(256)],
        out_shape=[
            jax.ShapeDtypeStruct((N, 256), jnp.float32),
            jax.ShapeDtypeStruct((N, 256), jnp.float32),
        ],
    )(d0, d1, x)

    xs2 = jnp.concatenate([xs[:, :128], xs[:, 128:]], axis=0)
    a1p = _agg_kernel(xs2, src_rows, dst_rows, z128)
    a1 = jnp.concatenate([a1p[:N], a1p[SP_ROWS:SP_ROWS + N]], axis=1)

    ts = pl.pallas_call(
        _mm_body,
        grid=grid,
        in_specs=[
            _row_spec(256), _row_spec(256), _row_spec(256),
            _full_spec(256, 512), _full_spec(1, 512), _full_spec(512, 256),
        ],
        out_specs=_row_spec(256),
        out_shape=jax.ShapeDtypeStruct((N, 256), jnp.float32),
    )(a1, xs, dis, W1, b1.reshape(1, 512), W2)

    ts2 = jnp.concatenate([ts[:, :128], ts[:, 128:]], axis=0)
    a2p = _agg_kernel(ts2, src_rows, dst_rows, z128)
    a2 = jnp.concatenate([a2p[:N], a2p[SP_ROWS:SP_ROWS + N]], axis=1)

    out = pl.pallas_call(
        _post_body,
        grid=grid,
        in_specs=[
            _row_spec(256), _row_spec(256), _row_spec(256), _full_spec(1, 256),
        ],
        out_specs=_row_spec(256),
        out_shape=jax.ShapeDtypeStruct((N, 256), jnp.float32),
    )(a2, ts, dis, b2.reshape(1, 256))
    return out


# speedup vs baseline: 2.2207x; 1.1524x over previous
import functools

import jax
import jax.numpy as jnp
from jax import lax
from jax.experimental import pallas as pl
from jax.experimental.pallas import tpu as pltpu
from jax.experimental.pallas import tpu_sc as plsc

N = 10000
E = 160000
CHUNK = 128
NC, NS = 2, 16
E_PAD = 163840
DUMP = 10000
ROWS_PER_TILE = 632
SP_ROWS = NS * ROWS_PER_TILE
DEG_CHUNKS = E_PAD // (NC * NS * CHUNK)
AGG_CHUNKS = E_PAD // (NS * CHUNK)

_mesh = plsc.VectorSubcoreMesh(core_axis_name="c", subcore_axis_name="s")


DEG_W = 128
DEG_NBUF = 4
NBUF = 2
IDX_AHEAD = 4
_DO_SCATTER = True


@functools.partial(
    pl.kernel,
    out_type=jax.ShapeDtypeStruct((NC * SP_ROWS, DEG_W), jnp.float32),
    mesh=_mesh,
    scratch_types=[
        pltpu.VMEM_SHARED((SP_ROWS, DEG_W), jnp.float32),
        pltpu.VMEM((DEG_CHUNKS, CHUNK), jnp.int32),
        pltpu.VMEM((CHUNK, DEG_W), jnp.float32),
        pltpu.SemaphoreType.DMA((DEG_NBUF,)),
    ],
)
def _deg_kernel(dst_hbm, ones_hbm, zeros_hbm, out_hbm, acc_sh, didx_all, ones_v, ssem):
    c = lax.axis_index("c")
    s = lax.axis_index("s")
    wid = c * NS + s
    pltpu.sync_copy(zeros_hbm, acc_sh.at[pl.ds(s * ROWS_PER_TILE, ROWS_PER_TILE)])
    pltpu.sync_copy(ones_hbm, ones_v)
    pltpu.sync_copy(dst_hbm.at[pl.ds(wid * DEG_CHUNKS, DEG_CHUNKS)], didx_all)
    plsc.subcore_barrier()

    sd = {}
    for k in range(DEG_CHUNKS):
        if k - DEG_NBUF >= 0:
            sd[k - DEG_NBUF].wait()
        sd[k] = pltpu.async_copy(
            ones_v, acc_sh.at[didx_all.at[k]], ssem.at[k % DEG_NBUF], add=True)
    for k in range(DEG_CHUNKS - DEG_NBUF, DEG_CHUNKS):
        sd[k].wait()
    plsc.subcore_barrier()
    pltpu.sync_copy(
        acc_sh.at[pl.ds(s * ROWS_PER_TILE, ROWS_PER_TILE)],
        out_hbm.at[pl.ds(c * SP_ROWS + s * ROWS_PER_TILE, ROWS_PER_TILE)],
    )


@functools.partial(
    pl.kernel,
    out_type=jax.ShapeDtypeStruct((NC * SP_ROWS, 128), jnp.float32),
    mesh=_mesh,
    scratch_types=[
        pltpu.VMEM_SHARED((SP_ROWS, 128), jnp.float32),
        pltpu.VMEM((AGG_CHUNKS, CHUNK), jnp.int32),
        [pltpu.VMEM((CHUNK,), jnp.int32) for _ in range(IDX_AHEAD)],
        pltpu.VMEM((NBUF * CHUNK, 128), jnp.float32),
        pltpu.SemaphoreType.DMA((IDX_AHEAD,)),
        pltpu.SemaphoreType.DMA((NBUF,)),
        pltpu.SemaphoreType.DMA((NBUF,)),
    ],
)
def _agg_kernel(xs_hbm, src_hbm, dst_hbm, zeros_hbm, out_hbm,
                acc_sh, didx_all, sidx, rows_v, isem, gsem, ssem):
    c = lax.axis_index("c")
    s = lax.axis_index("s")
    pltpu.sync_copy(zeros_hbm, acc_sh.at[pl.ds(s * ROWS_PER_TILE, ROWS_PER_TILE)])
    pltpu.sync_copy(dst_hbm.at[pl.ds(s * AGG_CHUNKS, AGG_CHUNKS)], didx_all)
    plsc.subcore_barrier()
    src_base = (c * NS + s) * AGG_CHUNKS

    def _gather(k):
        return pltpu.async_copy(
            xs_hbm.at[sidx[k % IDX_AHEAD]],
            rows_v.at[pl.ds((k % NBUF) * CHUNK, CHUNK)], gsem.at[k % NBUF])

    def _scatter(k):
        return pltpu.async_copy(
            rows_v.at[pl.ds((k % NBUF) * CHUNK, CHUNK)],
            acc_sh.at[didx_all.at[k]], ssem.at[k % NBUF], add=True)

    idn = {j: pltpu.async_copy(src_hbm.at[src_base + j], sidx[j % IDX_AHEAD],
                               isem.at[j % IDX_AHEAD])
           for j in range(IDX_AHEAD)}
    idn[0].wait()
    gd = {0: _gather(0)}
    sd = {}
    for k in range(AGG_CHUNKS):
        if k + 1 < AGG_CHUNKS:
            if k - 1 >= 0 and _DO_SCATTER:
                sd[k - 1].wait()
            idn[k + 1].wait()
            gd[k + 1] = _gather(k + 1)
        gd[k].wait()
        if _DO_SCATTER:
            sd[k] = _scatter(k)
        j = k + IDX_AHEAD
        if j < AGG_CHUNKS:
            idn[j] = pltpu.async_copy(src_hbm.at[src_base + j],
                                      sidx[j % IDX_AHEAD],
                                      isem.at[j % IDX_AHEAD])
    if _DO_SCATTER:
        sd[AGG_CHUNKS - 2].wait()
        sd[AGG_CHUNKS - 1].wait()
    plsc.subcore_barrier()
    pltpu.sync_copy(
        acc_sh.at[pl.ds(s * ROWS_PER_TILE, ROWS_PER_TILE)],
        out_hbm.at[pl.ds(c * SP_ROWS + s * ROWS_PER_TILE, ROWS_PER_TILE)],
    )


BLK = 1000


def _prep_body(dg0_ref, dg1_ref, x_ref, dis_ref, xs_ref):
    deg = dg0_ref[0, :, :1] + dg1_ref[0, :, :1] + 1.0
    dis = lax.rsqrt(deg)
    dis_ref[...] = jnp.broadcast_to(dis, dis_ref.shape)
    xs_ref[0] = x_ref[:, :128] * dis
    xs_ref[1] = x_ref[:, 128:] * dis


def _mm_body(a1l_ref, a1h_ref, xs_ref, dis_ref, w1_ref, b1_ref, w2_ref, ts_ref):
    dis = dis_ref[...]
    p = jnp.concatenate(
        [(a1l_ref[0] + xs_ref[0]) * dis, (a1h_ref[0] + xs_ref[1]) * dis], axis=1)
    h = jnp.dot(p, w1_ref[...], preferred_element_type=jnp.float32) + b1_ref[...]
    h = jnp.maximum(h, 0.0)
    t = jnp.dot(h, w2_ref[...], preferred_element_type=jnp.float32)
    ts_ref[0] = t[:, :128] * dis
    ts_ref[1] = t[:, 128:] * dis


def _post_body(a2l_ref, a2h_ref, ts_ref, dis_ref, b2_ref, out_ref):
    dis = dis_ref[...]
    out_ref[...] = jnp.concatenate(
        [(a2l_ref[0] + ts_ref[0]) * dis, (a2h_ref[0] + ts_ref[1]) * dis],
        axis=1) + b2_ref[...]


def _half_spec(h):
    return pl.BlockSpec((1, BLK, 128), lambda i, h=h: (h, i, 0))


def _pair_spec():
    return pl.BlockSpec((2, BLK, 128), lambda i: (0, i, 0))


def _row_spec(cols):
    return pl.BlockSpec((BLK, cols), lambda i: (i, 0))


def _full_spec(r, c):
    return pl.BlockSpec((r, c), lambda i: (0, 0))


def kernel(x, edge_index, W1, b1, W2, b2):
    src = edge_index[0].astype(jnp.int32)
    dst = edge_index[1].astype(jnp.int32)
    pad = E_PAD - E
    pad_idx = jnp.arange(pad, dtype=jnp.int32)
    src_p = jnp.concatenate([src, pad_idx % N])
    dst_p = jnp.concatenate([dst, DUMP + pad_idx % (SP_ROWS - DUMP)])
    dst_rows = dst_p.reshape(E_PAD // CHUNK, CHUNK)
    src_rows = jnp.concatenate([src_p, src_p + N]).reshape(2 * E_PAD // CHUNK, CHUNK)
    onesw = jnp.ones((CHUNK, DEG_W), jnp.float32)
    zw = jnp.zeros((ROWS_PER_TILE, DEG_W), jnp.float32)
    z128 = jnp.zeros((ROWS_PER_TILE, 128), jnp.float32)

    degp = _deg_kernel(dst_rows, onesw, zw).reshape(2, SP_ROWS, DEG_W)

    grid = (N // BLK,)
    dis, xs = pl.pallas_call(
        _prep_body,
        grid=grid,
        in_specs=[_half_spec(0), _half_spec(1), _row_spec(256)],
        out_specs=[_row_spec(128), _pair_spec()],
        out_shape=[
            jax.ShapeDtypeStruct((N, 128), jnp.float32),
            jax.ShapeDtypeStruct((2, N, 128), jnp.float32),
        ],
    )(degp, degp, x)

    a1p = _agg_kernel(xs.reshape(2 * N, 128), src_rows, dst_rows,
                      z128).reshape(2, SP_ROWS, 128)

    ts = pl.pallas_call(
        _mm_body,
        grid=grid,
        in_specs=[
            _half_spec(0), _half_spec(1), _pair_spec(), _row_spec(128),
            _full_spec(256, 512), _full_spec(1, 512), _full_spec(512, 256),
        ],
        out_specs=_pair_spec(),
        out_shape=jax.ShapeDtypeStruct((2, N, 128), jnp.float32),
    )(a1p, a1p, xs, dis, W1, b1.reshape(1, 512), W2)

    a2p = _agg_kernel(ts.reshape(2 * N, 128), src_rows, dst_rows,
                      z128).reshape(2, SP_ROWS, 128)

    out = pl.pallas_call(
        _post_body,
        grid=grid,
        in_specs=[
            _half_spec(0), _half_spec(1), _pair_spec(), _row_spec(128),
            _full_spec(1, 256),
        ],
        out_specs=_row_spec(256),
        out_shape=jax.ShapeDtypeStruct((N, 256), jnp.float32),
    )(a2p, a2p, ts, dis, b2.reshape(1, 256))
    return out
